# Initial kernel scaffold; baseline (speedup 1.0000x reference)
import jax
import jax.numpy as jnp
from jax.experimental import pallas as pl


def kernel(x, edge_index, batch, W1, b1, W2, b2, Wfc, bfc):
    raise NotImplementedError("write your pallas kernel here")


# speedup vs baseline: 21.9804x; 21.9804x over previous
import functools

import jax
import jax.numpy as jnp
from jax import lax
from jax.experimental import pallas as pl
from jax.experimental.pallas import tpu as pltpu
from jax.experimental.pallas import tpu_sc as plsc

N = 10000
E = 320000
D = 128
H = 128
C = 64
G = 64

NC = 2
NS = 16
L = 16
NW = NC * NS

CHUNK = 125
EROWS = E // CHUNK
RPT = EROWS // NW
EPT_DEG = E // NS
ROWS_PER_TILE = N // NS
DEG_ROWS = 640
DEG_RPT = DEG_ROWS // NS


def _newton_rsqrt(v):
    bits = plsc.bitcast(v, jnp.int32)
    seed = jnp.full((L,), 0x5F3759DF, jnp.int32) - (bits >> 1)
    y = plsc.bitcast(seed, jnp.float32)
    half = v * 0.5
    for _ in range(3):
        y = y * (1.5 - half * y * y)
    return y


def _deg_body(dst_hbm, rid_hbm, dinv_hbm, dstbuf, degloc, deg2d, rid_v, io_v,
              deg_sh):
    c = lax.axis_index("c")
    s = lax.axis_index("s")

    @pl.when(c == 0)
    def _():
        zero16 = jnp.zeros((L,), jnp.float32)

        def zloc(i, _):
            degloc[pl.ds(i * L, L)] = zero16
            io_v[i % DEG_RPT, :] = zero16
            return 0

        lax.fori_loop(0, DEG_ROWS, zloc, 0)
        pltpu.sync_copy(io_v, deg_sh.at[pl.ds(s * DEG_RPT, DEG_RPT)])

        pltpu.sync_copy(dst_hbm.at[pl.ds(s * EPT_DEG, EPT_DEG)], dstbuf)
        pltpu.sync_copy(rid_hbm, rid_v)

        ones16 = jnp.ones((L,), jnp.float32)

        def body(i, _):
            idx = dstbuf[pl.ds(i * L, L)]
            plsc.addupdate_scatter(degloc, [idx], ones16)
            return 0

        lax.fori_loop(0, EPT_DEG // L, body, 0)

        def repack(i, _):
            deg2d[i, :] = degloc[pl.ds(i * L, L)]
            return 0

        lax.fori_loop(0, DEG_ROWS, repack, 0)
        plsc.subcore_barrier()

        def comb(k, _):
            pltpu.sync_copy(deg2d.at[pl.ds(k * 125, 125)],
                            deg_sh.at[rid_v.at[k]], add=True)
            return 0

        lax.fori_loop(0, 5, comb, 0)
        plsc.subcore_barrier()

        pltpu.sync_copy(deg_sh.at[pl.ds(s * DEG_RPT, DEG_RPT)], io_v)

        def newt(r, _):
            io_v[r, :] = _newton_rsqrt(io_v[r, :] + 1.0)
            return 0

        lax.fori_loop(0, DEG_RPT, newt, 0)
        pltpu.sync_copy(io_v, dinv_hbm.at[pl.ds(s * DEG_RPT, DEG_RPT)])


def _sc_dinv(dst, row_ids):
    mesh = plsc.VectorSubcoreMesh(core_axis_name="c", subcore_axis_name="s")
    return pl.kernel(
        _deg_body,
        out_type=jax.ShapeDtypeStruct((DEG_ROWS, L), jnp.float32),
        mesh=mesh,
        compiler_params=pltpu.CompilerParams(needs_layout_passes=False),
        scratch_types=[
            pltpu.VMEM((EPT_DEG,), jnp.int32),
            pltpu.VMEM((DEG_ROWS * L,), jnp.float32),
            pltpu.VMEM((DEG_ROWS, L), jnp.float32),
            pltpu.VMEM((5, 125), jnp.int32),
            pltpu.VMEM((DEG_RPT, L), jnp.float32),
            pltpu.VMEM_SHARED((DEG_ROWS, L), jnp.float32),
        ],
    )(dst, row_ids)


HSLICE = 632
HLAST = N - (NS - 1) * HSLICE


def _agg_body(g_hbm, src_hbm, dst_hbm, out_hbm, src_v, dst_v, rows_v, acc_sh,
              gsem):
    c = lax.axis_index("c")
    s = lax.axis_index("s")
    base_row = pl.multiple_of((c * NS + s) * RPT, 8)
    rstart = pl.multiple_of(s * HSLICE, 8)

    @pl.when(s < NS - 1)
    def _():
        pltpu.sync_copy(g_hbm.at[pl.ds(rstart, HSLICE)],
                        acc_sh.at[pl.ds(rstart, HSLICE)])

    @pl.when(s == NS - 1)
    def _():
        pltpu.sync_copy(g_hbm.at[pl.ds((NS - 1) * HSLICE, HLAST)],
                        acc_sh.at[pl.ds((NS - 1) * HSLICE, HLAST)])

    pltpu.sync_copy(src_hbm.at[pl.ds(base_row, RPT)], src_v)
    pltpu.sync_copy(dst_hbm.at[pl.ds(base_row, RPT)], dst_v)
    plsc.subcore_barrier()

    def body(j, _):
        pltpu.async_copy(g_hbm.at[src_v.at[j]], rows_v, gsem).wait()
        pltpu.sync_copy(rows_v, acc_sh.at[dst_v.at[j]], add=True)
        return 0

    lax.fori_loop(0, RPT, body, 0)
    plsc.subcore_barrier()

    @pl.when(s < NS - 1)
    def _():
        pltpu.sync_copy(acc_sh.at[pl.ds(rstart, HSLICE)],
                        out_hbm.at[c, pl.ds(rstart, HSLICE)])

    @pl.when(s == NS - 1)
    def _():
        pltpu.sync_copy(acc_sh.at[pl.ds((NS - 1) * HSLICE, HLAST)],
                        out_hbm.at[c, pl.ds((NS - 1) * HSLICE, HLAST)])


def _sc_aggregate(g, src2d, dst2d):
    mesh = plsc.VectorSubcoreMesh(core_axis_name="c", subcore_axis_name="s")
    return pl.kernel(
        _agg_body,
        out_type=jax.ShapeDtypeStruct((NC, N, H), jnp.float32),
        mesh=mesh,
        compiler_params=pltpu.CompilerParams(needs_layout_passes=False),
        scratch_types=[
            pltpu.VMEM((RPT, CHUNK), jnp.int32),
            pltpu.VMEM((RPT, CHUNK), jnp.int32),
            pltpu.VMEM((CHUNK, H), jnp.float32),
            pltpu.VMEM_SHARED((N, H), jnp.float32),
            pltpu.SemaphoreType.DMA,
        ],
    )(g, src2d, dst2d)


BLK = 1000
NBLK = N // BLK


def _dense1_body(x_ref, w_ref, dinv_ref, g_ref):
    g_ref[...] = jnp.dot(x_ref[...], w_ref[...],
                         preferred_element_type=jnp.float32) * dinv_ref[...]


def _tc_dense1(x, W1, dinv):
    return pl.pallas_call(
        _dense1_body,
        grid=(NBLK,),
        in_specs=[
            pl.BlockSpec((BLK, D), lambda i: (i, 0)),
            pl.BlockSpec((D, H), lambda i: (0, 0)),
            pl.BlockSpec((BLK, 1), lambda i: (i, 0)),
        ],
        out_specs=pl.BlockSpec((BLK, H), lambda i: (i, 0)),
        out_shape=jax.ShapeDtypeStruct((N, H), jnp.float32),
    )(x, W1, dinv)


def _dense2_body(p0_ref, p1_ref, g1_ref, dinv_ref, b1_ref, w2_ref, g2_ref):
    agg = p0_ref[...] + p1_ref[...] - g1_ref[...]
    z = jnp.maximum(dinv_ref[...] * agg + b1_ref[...], 0.0)
    g2_ref[...] = jnp.dot(z, w2_ref[...],
                          preferred_element_type=jnp.float32) * dinv_ref[...]


def _tc_dense2(p0, p1, g1, dinv, b1, W2):
    return pl.pallas_call(
        _dense2_body,
        grid=(NBLK,),
        in_specs=[
            pl.BlockSpec((BLK, H), lambda i: (i, 0)),
            pl.BlockSpec((BLK, H), lambda i: (i, 0)),
            pl.BlockSpec((BLK, H), lambda i: (i, 0)),
            pl.BlockSpec((BLK, 1), lambda i: (i, 0)),
            pl.BlockSpec((1, H), lambda i: (0, 0)),
            pl.BlockSpec((H, H), lambda i: (0, 0)),
        ],
        out_specs=pl.BlockSpec((BLK, H), lambda i: (i, 0)),
        out_shape=jax.ShapeDtypeStruct((N, H), jnp.float32),
    )(p0, p1, g1, dinv, b1, W2)


def _dense3_body(p0_ref, p1_ref, g2_ref, dinv_ref, b2_ref, batch_ref, wfc_ref,
                 bfc_ref, out_ref, acc_s, cnt_s):
    i = pl.program_id(0)
    agg = p0_ref[...] + p1_ref[...] - g2_ref[...]
    z = jnp.maximum(dinv_ref[...] * agg + b2_ref[...], 0.0)
    bt = batch_ref[...].reshape(1, BLK)
    oneh = (lax.broadcasted_iota(jnp.int32, (G, BLK), 0) == bt
            ).astype(jnp.float32)
    part = jnp.dot(oneh, z, preferred_element_type=jnp.float32)
    cnt = jnp.dot(oneh, jnp.ones((BLK, H), jnp.float32),
                  preferred_element_type=jnp.float32)

    @pl.when(i == 0)
    def _():
        acc_s[...] = part
        cnt_s[...] = cnt

    @pl.when(i > 0)
    def _():
        acc_s[...] += part
        cnt_s[...] += cnt

    @pl.when(i == NBLK - 1)
    def _():
        pooled = acc_s[...] / jnp.maximum(cnt_s[...], 1.0)
        out_ref[...] = jnp.dot(pooled, wfc_ref[...],
                               preferred_element_type=jnp.float32) + bfc_ref[...]


def _tc_dense3(p0, p1, g2, dinv, b2, batch3, Wfc, bfc):
    return pl.pallas_call(
        _dense3_body,
        grid=(NBLK,),
        in_specs=[
            pl.BlockSpec((BLK, H), lambda i: (i, 0)),
            pl.BlockSpec((BLK, H), lambda i: (i, 0)),
            pl.BlockSpec((BLK, H), lambda i: (i, 0)),
            pl.BlockSpec((BLK, 1), lambda i: (i, 0)),
            pl.BlockSpec((1, H), lambda i: (0, 0)),
            pl.BlockSpec((1, 1, BLK), lambda i: (i, 0, 0)),
            pl.BlockSpec((H, C), lambda i: (0, 0)),
            pl.BlockSpec((1, C), lambda i: (0, 0)),
        ],
        out_specs=pl.BlockSpec((G, C), lambda i: (0, 0)),
        out_shape=jax.ShapeDtypeStruct((G, C), jnp.float32),
        scratch_shapes=[
            pltpu.VMEM((G, H), jnp.float32),
            pltpu.VMEM((G, H), jnp.float32),
        ],
    )(p0, p1, g2, dinv, b2, batch3, Wfc, bfc)


def kernel(x, edge_index, batch, W1, b1, W2, b2, Wfc, bfc):
    src = edge_index[0]
    dst = edge_index[1]
    src2d = src.reshape(EROWS, CHUNK)
    dst2d = dst.reshape(EROWS, CHUNK)
    row_ids = jnp.arange(625, dtype=jnp.int32).reshape(5, 125)

    dinv_flat = _sc_dinv(dst, row_ids)
    dinv = dinv_flat.reshape(-1)[:N].reshape(N, 1)

    g1 = _tc_dense1(x, W1, dinv)
    parts1 = _sc_aggregate(g1, src2d, dst2d)
    g2 = _tc_dense2(parts1[0], parts1[1], g1, dinv, b1.reshape(1, H), W2)
    parts2 = _sc_aggregate(g2, src2d, dst2d)

    batch3 = batch.reshape(NBLK, 1, BLK)
    return _tc_dense3(parts2[0], parts2[1], g2, dinv, b2.reshape(1, H),
                      batch3, Wfc, bfc.reshape(1, C))


# speedup vs baseline: 27.7932x; 1.2645x over previous
import jax
import jax.numpy as jnp
from jax import lax
from jax.experimental import pallas as pl
from jax.experimental.pallas import tpu as pltpu
from jax.experimental.pallas import tpu_sc as plsc

N = 10000
E = 320000
D = 128
H = 128
C = 64
G = 64

NC = 2
NS = 16
L = 16
NW = NC * NS

CHUNK = 125
EROWS = E // CHUNK
RPT = EROWS // NW
GRP = 16
NGRP = RPT // GRP
EPT_DEG = E // NS
DEG_ROWS = 640
DEG_RPT = DEG_ROWS // NS

HSLICE = 632
HLAST = N - (NS - 1) * HSLICE


def _newton_rsqrt(v):
    bits = plsc.bitcast(v, jnp.int32)
    seed = jnp.full((L,), 0x5F3759DF, jnp.int32) - (bits >> 1)
    y = plsc.bitcast(seed, jnp.float32)
    half = v * 0.5
    for _ in range(3):
        y = y * (1.5 - half * y * y)
    return y


def _deg_body(dst_hbm, rid_hbm, dinv_hbm, dstbuf, degloc, deg2d, rid_v, io_v,
              deg_sh):
    c = lax.axis_index("c")
    s = lax.axis_index("s")

    @pl.when(c == 0)
    def _():
        zero16 = jnp.zeros((L,), jnp.float32)

        def zloc(i, _):
            degloc[pl.ds(i * L, L)] = zero16
            io_v[i % DEG_RPT, :] = zero16
            return 0

        lax.fori_loop(0, DEG_ROWS, zloc, 0)
        pltpu.sync_copy(io_v, deg_sh.at[pl.ds(s * DEG_RPT, DEG_RPT)])

        pltpu.sync_copy(dst_hbm.at[pl.ds(s * EPT_DEG, EPT_DEG)], dstbuf)
        pltpu.sync_copy(rid_hbm, rid_v)

        ones16 = jnp.ones((L,), jnp.float32)

        def body(i, _):
            idx = dstbuf[pl.ds(i * L, L)]
            plsc.addupdate_scatter(degloc, [idx], ones16)
            return 0

        lax.fori_loop(0, EPT_DEG // L, body, 0)

        def repack(i, _):
            deg2d[i, :] = degloc[pl.ds(i * L, L)]
            return 0

        lax.fori_loop(0, DEG_ROWS, repack, 0)
        plsc.subcore_barrier()

        def comb(k, _):
            pltpu.sync_copy(deg2d.at[pl.ds(k * 125, 125)],
                            deg_sh.at[rid_v.at[k]], add=True)
            return 0

        lax.fori_loop(0, 5, comb, 0)
        plsc.subcore_barrier()

        pltpu.sync_copy(deg_sh.at[pl.ds(s * DEG_RPT, DEG_RPT)], io_v)

        def newt(r, _):
            io_v[r, :] = _newton_rsqrt(io_v[r, :] + 1.0)
            return 0

        lax.fori_loop(0, DEG_RPT, newt, 0)
        pltpu.sync_copy(io_v, dinv_hbm.at[pl.ds(s * DEG_RPT, DEG_RPT)])


def _sc_dinv(dst, row_ids):
    mesh = plsc.VectorSubcoreMesh(core_axis_name="c", subcore_axis_name="s")
    return pl.kernel(
        _deg_body,
        out_type=jax.ShapeDtypeStruct((DEG_ROWS, L), jnp.float32),
        mesh=mesh,
        compiler_params=pltpu.CompilerParams(needs_layout_passes=False),
        scratch_types=[
            pltpu.VMEM((EPT_DEG,), jnp.int32),
            pltpu.VMEM((DEG_ROWS * L,), jnp.float32),
            pltpu.VMEM((DEG_ROWS, L), jnp.float32),
            pltpu.VMEM((5, 125), jnp.int32),
            pltpu.VMEM((DEG_RPT, L), jnp.float32),
            pltpu.VMEM_SHARED((DEG_ROWS, L), jnp.float32),
        ],
    )(dst, row_ids)


def _agg_body(g_hbm, src_hbm, dst_hbm, out_hbm, src_v, dst_v, rows_a, rows_b,
              acc_sh, sem_a, sem_b):
    c = lax.axis_index("c")
    s = lax.axis_index("s")
    base_row = pl.multiple_of((c * NS + s) * RPT, 8)
    rstart = pl.multiple_of(s * HSLICE, 8)

    @pl.when(s < NS - 1)
    def _():
        pltpu.sync_copy(g_hbm.at[pl.ds(rstart, HSLICE)],
                        acc_sh.at[pl.ds(rstart, HSLICE)])

    @pl.when(s == NS - 1)
    def _():
        pltpu.sync_copy(g_hbm.at[pl.ds((NS - 1) * HSLICE, HLAST)],
                        acc_sh.at[pl.ds((NS - 1) * HSLICE, HLAST)])

    pltpu.sync_copy(src_hbm.at[pl.ds(base_row, RPT)], src_v)
    plsc.subcore_barrier()

    def group(gi, _):
        pltpu.sync_copy(dst_hbm.at[pl.ds(base_row + gi * GRP, GRP)], dst_v)
        g0 = gi * GRP
        pltpu.async_copy(g_hbm.at[src_v.at[g0]], rows_a, sem_a)

        def pair(j2, _):
            j = g0 + j2 * 2
            jj = j2 * 2
            pltpu.make_async_copy(g_hbm.at[src_v.at[j]], rows_a, sem_a).wait()
            pltpu.async_copy(g_hbm.at[src_v.at[j + 1]], rows_b, sem_b)
            pltpu.sync_copy(rows_a, acc_sh.at[dst_v.at[jj]], add=True)
            pltpu.make_async_copy(g_hbm.at[src_v.at[j + 1]], rows_b,
                                  sem_b).wait()

            @pl.when(j2 < GRP // 2 - 1)
            def _():
                pltpu.async_copy(g_hbm.at[src_v.at[j + 2]], rows_a, sem_a)

            pltpu.sync_copy(rows_b, acc_sh.at[dst_v.at[jj + 1]], add=True)
            return 0

        lax.fori_loop(0, GRP // 2, pair, 0)
        return 0

    lax.fori_loop(0, NGRP, group, 0)
    plsc.subcore_barrier()

    @pl.when(s < NS - 1)
    def _():
        pltpu.sync_copy(acc_sh.at[pl.ds(rstart, HSLICE)],
                        out_hbm.at[c, pl.ds(rstart, HSLICE)])

    @pl.when(s == NS - 1)
    def _():
        pltpu.sync_copy(acc_sh.at[pl.ds((NS - 1) * HSLICE, HLAST)],
                        out_hbm.at[c, pl.ds((NS - 1) * HSLICE, HLAST)])


def _sc_aggregate(g, src2d, dst2d):
    mesh = plsc.VectorSubcoreMesh(core_axis_name="c", subcore_axis_name="s")
    return pl.kernel(
        _agg_body,
        out_type=jax.ShapeDtypeStruct((NC, N, H), jnp.float32),
        mesh=mesh,
        compiler_params=pltpu.CompilerParams(needs_layout_passes=False),
        scratch_types=[
            pltpu.VMEM((RPT, CHUNK), jnp.int32),
            pltpu.VMEM((GRP, CHUNK), jnp.int32),
            pltpu.VMEM((CHUNK, H), jnp.float32),
            pltpu.VMEM((CHUNK, H), jnp.float32),
            pltpu.VMEM_SHARED((N, H), jnp.float32),
            pltpu.SemaphoreType.DMA,
            pltpu.SemaphoreType.DMA,
        ],
    )(g, src2d, dst2d)


BLK = 1000
NBLK = N // BLK


def _dense1_body(x_ref, w_ref, dinv_ref, g_ref):
    g_ref[...] = jnp.dot(x_ref[...], w_ref[...],
                         preferred_element_type=jnp.float32) * dinv_ref[...]


def _tc_dense1(x, W1, dinv):
    return pl.pallas_call(
        _dense1_body,
        grid=(NBLK,),
        in_specs=[
            pl.BlockSpec((BLK, D), lambda i: (i, 0)),
            pl.BlockSpec((D, H), lambda i: (0, 0)),
            pl.BlockSpec((BLK, 1), lambda i: (i, 0)),
        ],
        out_specs=pl.BlockSpec((BLK, H), lambda i: (i, 0)),
        out_shape=jax.ShapeDtypeStruct((N, H), jnp.float32),
    )(x, W1, dinv)


def _dense2_body(p0_ref, p1_ref, g1_ref, dinv_ref, b1_ref, w2_ref, g2_ref):
    agg = p0_ref[0] + p1_ref[0] - g1_ref[...]
    z = jnp.maximum(dinv_ref[...] * agg + b1_ref[...], 0.0)
    g2_ref[...] = jnp.dot(z, w2_ref[...],
                          preferred_element_type=jnp.float32) * dinv_ref[...]


def _tc_dense2(p, g1, dinv, b1, W2):
    return pl.pallas_call(
        _dense2_body,
        grid=(NBLK,),
        in_specs=[
            pl.BlockSpec((1, BLK, H), lambda i: (0, i, 0)),
            pl.BlockSpec((1, BLK, H), lambda i: (1, i, 0)),
            pl.BlockSpec((BLK, H), lambda i: (i, 0)),
            pl.BlockSpec((BLK, 1), lambda i: (i, 0)),
            pl.BlockSpec((1, H), lambda i: (0, 0)),
            pl.BlockSpec((H, H), lambda i: (0, 0)),
        ],
        out_specs=pl.BlockSpec((BLK, H), lambda i: (i, 0)),
        out_shape=jax.ShapeDtypeStruct((N, H), jnp.float32),
    )(p, p, g1, dinv, b1, W2)


def _dense3_body(p0_ref, p1_ref, g2_ref, dinv_ref, b2_ref, batch_ref, wfc_ref,
                 bfc_ref, out_ref, acc_s, cnt_s):
    i = pl.program_id(0)
    agg = p0_ref[0] + p1_ref[0] - g2_ref[...]
    z = jnp.maximum(dinv_ref[...] * agg + b2_ref[...], 0.0)
    bt = batch_ref[...].reshape(1, BLK)
    oneh = (lax.broadcasted_iota(jnp.int32, (G, BLK), 0) == bt
            ).astype(jnp.float32)
    part = jnp.dot(oneh, z, preferred_element_type=jnp.float32)
    cnt = jnp.dot(oneh, jnp.ones((BLK, H), jnp.float32),
                  preferred_element_type=jnp.float32)

    @pl.when(i == 0)
    def _():
        acc_s[...] = part
        cnt_s[...] = cnt

    @pl.when(i > 0)
    def _():
        acc_s[...] += part
        cnt_s[...] += cnt

    @pl.when(i == NBLK - 1)
    def _():
        pooled = acc_s[...] / jnp.maximum(cnt_s[...], 1.0)
        out_ref[...] = jnp.dot(pooled, wfc_ref[...],
                               preferred_element_type=jnp.float32) + bfc_ref[...]


def _tc_dense3(p, g2, dinv, b2, batch3, Wfc, bfc):
    return pl.pallas_call(
        _dense3_body,
        grid=(NBLK,),
        in_specs=[
            pl.BlockSpec((1, BLK, H), lambda i: (0, i, 0)),
            pl.BlockSpec((1, BLK, H), lambda i: (1, i, 0)),
            pl.BlockSpec((BLK, H), lambda i: (i, 0)),
            pl.BlockSpec((BLK, 1), lambda i: (i, 0)),
            pl.BlockSpec((1, H), lambda i: (0, 0)),
            pl.BlockSpec((1, 1, BLK), lambda i: (i, 0, 0)),
            pl.BlockSpec((H, C), lambda i: (0, 0)),
            pl.BlockSpec((1, C), lambda i: (0, 0)),
        ],
        out_specs=pl.BlockSpec((G, C), lambda i: (0, 0)),
        out_shape=jax.ShapeDtypeStruct((G, C), jnp.float32),
        scratch_shapes=[
            pltpu.VMEM((G, H), jnp.float32),
            pltpu.VMEM((G, H), jnp.float32),
        ],
    )(p, p, g2, dinv, b2, batch3, Wfc, bfc)


def kernel(x, edge_index, batch, W1, b1, W2, b2, Wfc, bfc):
    src = edge_index[0]
    dst = edge_index[1]
    src2d = src.reshape(EROWS, CHUNK)
    dst2d = dst.reshape(EROWS, CHUNK)
    row_ids = jnp.arange(625, dtype=jnp.int32).reshape(5, 125)

    dinv_flat = _sc_dinv(dst, row_ids)
    dinv = dinv_flat.reshape(-1)[:N].reshape(N, 1)

    g1 = _tc_dense1(x, W1, dinv)
    p1 = _sc_aggregate(g1, src2d, dst2d)
    g2 = _tc_dense2(p1, g1, dinv, b1.reshape(1, H), W2)
    p2 = _sc_aggregate(g2, src2d, dst2d)

    batch3 = batch.reshape(NBLK, 1, BLK)
    return _tc_dense3(p2, g2, dinv, b2.reshape(1, H), batch3, Wfc,
                      bfc.reshape(1, C))
